# baseline (device time: 21557 ns/iter reference)
import jax
import jax.numpy as jnp
from jax import lax
from jax.experimental import pallas as pl
from jax.experimental.pallas import tpu as pltpu

N_CHUNK = 4

_GATE_DIMS = (((1,), (1,)), ((), ()))


def kernel(x, router, W1, W2):
    t_per, d = x.shape
    e_per = W1.shape[0]
    router_t = router.T
    rows = t_per // N_CHUNK

    def body(x_ref, rt_ref, W1_ref, W2_ref, out_ref,
             xsend_ref, xrecv_ref, rrecv_ref, wsend_ref, wrecv_ref,
             psend_ref, cbuf_ref, send_sems, recv_sems):
        my_x = lax.axis_index("x")
        my_y = lax.axis_index("y")
        my_z = lax.axis_index("z")
        partner = (my_x, my_y, 1 - my_z)

        barrier_sem = pltpu.get_barrier_semaphore()
        pl.semaphore_signal(barrier_sem, inc=1, device_id=partner,
                            device_id_type=pl.DeviceIdType.MESH)
        pl.semaphore_wait(barrier_sem, 1)

        rdma_r = pltpu.make_async_remote_copy(
            src_ref=rt_ref, dst_ref=rrecv_ref,
            send_sem=send_sems.at[1], recv_sem=recv_sems.at[1],
            device_id=partner, device_id_type=pl.DeviceIdType.MESH)
        rdma_r.start()
        xsend_ref[...] = x_ref[...].astype(jnp.bfloat16)
        rdma_x = pltpu.make_async_remote_copy(
            src_ref=xsend_ref, dst_ref=xrecv_ref,
            send_sem=send_sems.at[0], recv_sem=recv_sems.at[0],
            device_id=partner, device_id_type=pl.DeviceIdType.MESH)
        rdma_x.start()

        def topk_weights(a0, a1, b0, b1):
            ma, sa = jnp.maximum(a0, a1), jnp.minimum(a0, a1)
            mb, sb = jnp.maximum(b0, b1), jnp.minimum(b0, b1)
            m1 = jnp.maximum(ma, mb)
            m2 = jnp.where(ma >= mb, jnp.maximum(sa, mb), jnp.maximum(sb, ma))
            t1 = 1.0 / (1.0 + jnp.exp(m2 - m1))
            t2 = 1.0 - t1

            def wexp(g):
                return jnp.where(g == m1, t1, jnp.where(g == m2, t2, 0.0))

            return wexp(a0), wexp(a1)

        def ffn(xs):
            h0 = jnp.maximum(
                jnp.dot(xs, W1_ref[0], preferred_element_type=jnp.float32), 0.0)
            o0 = jnp.dot(h0, W2_ref[0], preferred_element_type=jnp.float32)
            h1 = jnp.maximum(
                jnp.dot(xs, W1_ref[1], preferred_element_type=jnp.float32), 0.0)
            o1 = jnp.dot(h1, W2_ref[1], preferred_element_type=jnp.float32)
            return o0, o1

        xs = x_ref[...]
        g_mine = lax.dot_general(xs, rt_ref[...], _GATE_DIMS,
                                 preferred_element_type=jnp.float32)
        o0, o1 = ffn(xs)
        rdma_r.wait()
        g_oth = lax.dot_general(xs, rrecv_ref[...], _GATE_DIMS,
                                preferred_element_type=jnp.float32)
        w0, w1 = topk_weights(g_mine[:, 0:1], g_mine[:, 1:2],
                              g_oth[:, 0:1], g_oth[:, 1:2])

        gt_mine = lax.dot_general(rt_ref[...], xs, _GATE_DIMS,
                                  preferred_element_type=jnp.float32)
        gt_oth = lax.dot_general(rrecv_ref[...], xs, _GATE_DIMS,
                                 preferred_element_type=jnp.float32)
        wt0, wt1 = topk_weights(gt_oth[0:1, :], gt_oth[1:2, :],
                                gt_mine[0:1, :], gt_mine[1:2, :])
        wsend_ref[0:1, :] = wt0
        wsend_ref[1:2, :] = wt1
        rdma_w = pltpu.make_async_remote_copy(
            src_ref=wsend_ref, dst_ref=wrecv_ref,
            send_sem=send_sems.at[2], recv_sem=recv_sems.at[2],
            device_id=partner, device_id_type=pl.DeviceIdType.MESH)
        rdma_w.start()

        rdma_x.wait()
        xp = xrecv_ref[...].astype(jnp.float32)
        rdma_w.wait()
        rdma_p = []
        for c in range(N_CHUNK):
            lo = c * rows
            oc0, oc1 = ffn(xp[lo:lo + rows])
            row_i = lax.broadcasted_iota(jnp.int32, (rows, rows), 0)
            col_i = lax.broadcasted_iota(jnp.int32, (rows, rows), 1)
            d0 = jnp.where(row_i == col_i, wrecv_ref[0:1, lo:lo + rows], 0.0)
            d1 = jnp.where(row_i == col_i, wrecv_ref[1:2, lo:lo + rows], 0.0)
            pc = (jnp.dot(d0, oc0, preferred_element_type=jnp.float32)
                  + jnp.dot(d1, oc1, preferred_element_type=jnp.float32))
            psend_ref[c] = pc.astype(jnp.bfloat16)
            r = pltpu.make_async_remote_copy(
                src_ref=psend_ref.at[c], dst_ref=cbuf_ref.at[c],
                send_sem=send_sems.at[3 + c], recv_sem=recv_sems.at[3 + c],
                device_id=partner, device_id_type=pl.DeviceIdType.MESH)
            r.start()
            rdma_p.append(r)

        out_ref[...] = o0 * w0 + o1 * w1

        for c in range(N_CHUNK):
            rdma_p[c].wait()
            lo = c * rows
            out_ref[lo:lo + rows, :] = (out_ref[lo:lo + rows, :]
                                        + cbuf_ref[c].astype(jnp.float32))

    out_shape = jax.ShapeDtypeStruct((t_per, d), jnp.float32)
    return pl.pallas_call(
        body,
        out_shape=out_shape,
        in_specs=[pl.BlockSpec(memory_space=pltpu.VMEM)] * 4,
        out_specs=pl.BlockSpec(memory_space=pltpu.VMEM),
        scratch_shapes=[
            pltpu.VMEM((t_per, d), jnp.bfloat16),
            pltpu.VMEM((t_per, d), jnp.bfloat16),
            pltpu.VMEM((e_per, d), jnp.float32),
            pltpu.VMEM((e_per, t_per), jnp.float32),
            pltpu.VMEM((e_per, t_per), jnp.float32),
            pltpu.VMEM((N_CHUNK, rows, d), jnp.bfloat16),
            pltpu.VMEM((N_CHUNK, rows, d), jnp.bfloat16),
            pltpu.SemaphoreType.DMA((3 + N_CHUNK,)),
            pltpu.SemaphoreType.DMA((3 + N_CHUNK,)),
        ],
        compiler_params=pltpu.CompilerParams(collective_id=0),
    )(x, router_t, W1, W2)


# device time: 20047 ns/iter; 1.0753x vs baseline; 1.0753x over previous
import jax
import jax.numpy as jnp
from jax import lax
from jax.experimental import pallas as pl
from jax.experimental.pallas import tpu as pltpu

CHUNKS = ((0, 128), (128, 128))
N_CHUNK = len(CHUNKS)

_GATE_DIMS = (((1,), (1,)), ((), ()))


def kernel(x, router, W1, W2):
    t_per, d = x.shape
    e_per = W1.shape[0]
    router_t = router.T

    def body(x_ref, rt_ref, W1_ref, W2_ref, out_ref,
             xsend_ref, xrecv_ref, rrecv_ref, wsend_ref, wrecv_ref,
             psend_ref, cbuf_ref, send_sems, recv_sems):
        my_x = lax.axis_index("x")
        my_y = lax.axis_index("y")
        my_z = lax.axis_index("z")
        partner = (my_x, my_y, 1 - my_z)

        barrier_sem = pltpu.get_barrier_semaphore()
        pl.semaphore_signal(barrier_sem, inc=1, device_id=partner,
                            device_id_type=pl.DeviceIdType.MESH)
        pl.semaphore_wait(barrier_sem, 1)

        rdma_r = pltpu.make_async_remote_copy(
            src_ref=rt_ref, dst_ref=rrecv_ref,
            send_sem=send_sems.at[1], recv_sem=recv_sems.at[1],
            device_id=partner, device_id_type=pl.DeviceIdType.MESH)
        rdma_r.start()
        xsend_ref[...] = x_ref[...].astype(jnp.bfloat16)
        rdma_x = pltpu.make_async_remote_copy(
            src_ref=xsend_ref, dst_ref=xrecv_ref,
            send_sem=send_sems.at[0], recv_sem=recv_sems.at[0],
            device_id=partner, device_id_type=pl.DeviceIdType.MESH)
        rdma_x.start()

        def topk_weights(a0, a1, b0, b1):
            ma, sa = jnp.maximum(a0, a1), jnp.minimum(a0, a1)
            mb, sb = jnp.maximum(b0, b1), jnp.minimum(b0, b1)
            m1 = jnp.maximum(ma, mb)
            m2 = jnp.where(ma >= mb, jnp.maximum(sa, mb), jnp.maximum(sb, ma))
            t1 = 1.0 / (1.0 + jnp.exp(m2 - m1))
            t2 = 1.0 - t1

            def wexp(g):
                return jnp.where(g == m1, t1, jnp.where(g == m2, t2, 0.0))

            return wexp(a0), wexp(a1)

        def ffn(xs):
            h0 = jnp.maximum(
                jnp.dot(xs, W1_ref[0], preferred_element_type=jnp.float32), 0.0)
            o0 = jnp.dot(h0, W2_ref[0], preferred_element_type=jnp.float32)
            h1 = jnp.maximum(
                jnp.dot(xs, W1_ref[1], preferred_element_type=jnp.float32), 0.0)
            o1 = jnp.dot(h1, W2_ref[1], preferred_element_type=jnp.float32)
            return o0, o1

        xs = x_ref[...]
        g_mine = lax.dot_general(xs, rt_ref[...], _GATE_DIMS,
                                 preferred_element_type=jnp.float32)
        h0 = jnp.maximum(
            jnp.dot(xs, W1_ref[0], preferred_element_type=jnp.float32), 0.0)
        o0 = jnp.dot(h0, W2_ref[0], preferred_element_type=jnp.float32)
        h1 = jnp.maximum(
            jnp.dot(xs, W1_ref[1], preferred_element_type=jnp.float32), 0.0)
        rdma_r.wait()
        g_oth = lax.dot_general(xs, rrecv_ref[...], _GATE_DIMS,
                                preferred_element_type=jnp.float32)
        w0, w1 = topk_weights(g_mine[:, 0:1], g_mine[:, 1:2],
                              g_oth[:, 0:1], g_oth[:, 1:2])

        wt0, wt1 = topk_weights(g_oth[:, 0:1], g_oth[:, 1:2],
                                g_mine[:, 0:1], g_mine[:, 1:2])
        wsend_ref[...] = jnp.concatenate([wt0, wt1], axis=1).T
        rdma_w = pltpu.make_async_remote_copy(
            src_ref=wsend_ref, dst_ref=wrecv_ref,
            send_sem=send_sems.at[2], recv_sem=recv_sems.at[2],
            device_id=partner, device_id_type=pl.DeviceIdType.MESH)
        rdma_w.start()

        rdma_x.wait()
        xp = xrecv_ref[...].astype(jnp.float32)
        rdma_w.wait()
        wp = wrecv_ref[...].T
        rdma_p = []
        for c, (lo, n) in enumerate(CHUNKS):
            oc0, oc1 = ffn(xp[lo:lo + n])
            pc = (oc0 * wp[lo:lo + n, 0:1] + oc1 * wp[lo:lo + n, 1:2])
            psend_ref[lo:lo + n, :] = pc.astype(jnp.bfloat16)
            r = pltpu.make_async_remote_copy(
                src_ref=psend_ref.at[pl.ds(lo, n)],
                dst_ref=cbuf_ref.at[pl.ds(lo, n)],
                send_sem=send_sems.at[3 + c], recv_sem=recv_sems.at[3 + c],
                device_id=partner, device_id_type=pl.DeviceIdType.MESH)
            r.start()
            rdma_p.append(r)

        o1 = jnp.dot(h1, W2_ref[1], preferred_element_type=jnp.float32)
        out_ref[...] = o0 * w0 + o1 * w1

        for c, (lo, n) in enumerate(CHUNKS):
            rdma_p[c].wait()
            out_ref[lo:lo + n, :] = (out_ref[lo:lo + n, :]
                                     + cbuf_ref[lo:lo + n, :].astype(jnp.float32))

    out_shape = jax.ShapeDtypeStruct((t_per, d), jnp.float32)
    return pl.pallas_call(
        body,
        out_shape=out_shape,
        in_specs=[pl.BlockSpec(memory_space=pltpu.VMEM)] * 4,
        out_specs=pl.BlockSpec(memory_space=pltpu.VMEM),
        scratch_shapes=[
            pltpu.VMEM((t_per, d), jnp.bfloat16),
            pltpu.VMEM((t_per, d), jnp.bfloat16),
            pltpu.VMEM((e_per, d), jnp.float32),
            pltpu.VMEM((e_per, t_per), jnp.float32),
            pltpu.VMEM((e_per, t_per), jnp.float32),
            pltpu.VMEM((t_per, d), jnp.bfloat16),
            pltpu.VMEM((t_per, d), jnp.bfloat16),
            pltpu.SemaphoreType.DMA((3 + N_CHUNK,)),
            pltpu.SemaphoreType.DMA((3 + N_CHUNK,)),
        ],
        compiler_params=pltpu.CompilerParams(collective_id=0),
    )(x, router_t, W1, W2)
